# baseline (speedup 1.0000x reference)
import functools

import jax
import jax.numpy as jnp
from jax import lax
from jax.experimental import pallas as pl
from jax.experimental.pallas import tpu as pltpu
from jax.experimental.pallas import tpu_sc as plsc

NC = 2
NS = 16
NW = NC * NS
LANES = 16
C = 125
G = 16


def _tc_pre_body(T, x_ref, nt_ref, fW_ref, fb_ref, W0_ref, out_ref):
  xb = x_ref[...]
  nt = nt_ref[...]
  h = jnp.zeros_like(xb)
  for t in range(T):
    ht = jnp.dot(xb, fW_ref[t], preferred_element_type=jnp.float32)
    ht = ht + fb_ref[t]
    h = jnp.where(nt == t, ht, h)
  out_ref[...] = jnp.dot(h, W0_ref[...], preferred_element_type=jnp.float32)


def _tc_pre(x, node_type, f_W, f_b, W0, NP, block):
  N, D = x.shape
  T = f_W.shape[0]
  grid = (NP // block,)
  return pl.pallas_call(
      functools.partial(_tc_pre_body, T),
      grid=grid,
      in_specs=[
          pl.BlockSpec((block, D), lambda i: (i, 0)),
          pl.BlockSpec((block, 1), lambda i: (i, 0)),
          pl.BlockSpec((T, D, D), lambda i: (0, 0, 0)),
          pl.BlockSpec((T, 1, D), lambda i: (0, 0, 0)),
          pl.BlockSpec((D, D), lambda i: (0, 0)),
      ],
      out_specs=pl.BlockSpec((block, D), lambda i: (i, 0)),
      out_shape=jax.ShapeDtypeStruct((NP, D), jnp.float32),
  )(x, node_type.reshape(N, 1), f_W, f_b.reshape(T, 1, D), W0)


def _tc_post_body(has_W, a0_ref, a1_ref, dinv_ref, b_ref, W_ref, out_ref):
  agg = a0_ref[...] + a1_ref[...]
  h = jnp.maximum(agg * dinv_ref[...] + b_ref[...], 0.0)
  if has_W:
    h = jnp.dot(h, W_ref[...], preferred_element_type=jnp.float32)
  out_ref[...] = h


def _tc_post(acc, dinv, b, W_next, block):
  NP2, D = acc.shape
  NP = NP2 // 2
  grid = (NP // block,)
  nblk = NP // block
  has_W = W_next is not None
  if W_next is None:
    W_next = jnp.zeros((8, 128), jnp.float32)
  WD = W_next.shape[0]
  return pl.pallas_call(
      functools.partial(_tc_post_body, has_W),
      grid=grid,
      in_specs=[
          pl.BlockSpec((block, D), lambda i: (i, 0)),
          pl.BlockSpec((block, D), lambda i: (i + nblk, 0)),
          pl.BlockSpec((block, 1), lambda i: (i, 0)),
          pl.BlockSpec((1, D), lambda i: (0, 0)),
          pl.BlockSpec((WD, D), lambda i: (0, 0)),
      ],
      out_specs=pl.BlockSpec((block, D), lambda i: (i, 0)),
      out_shape=jax.ShapeDtypeStruct((NP, D), jnp.float32),
  )(acc, acc, dinv, b.reshape(1, D), W_next)


def _sc_edge_body(NP, D, NCH, rows_per_tile, compute_deg,
                  g_hbm, ei_hbm, z_hbm, z1_hbm,
                  *refs):
  if compute_deg:
    (acc_out, deg_out, src_sv, dst_sv, rows_v0, rows_v1, acc_sh, deg_v,
     semg0, semg1, sems0, sems1) = refs
  else:
    (acc_out, src_sv, dst_sv, rows_v0, rows_v1, acc_sh,
     semg0, semg1, sems0, sems1) = refs
    deg_out = deg_v = None
  rows = (rows_v0, rows_v1)
  semg = (semg0, semg1)
  sems = (sems0, sems1)
  cid = lax.axis_index("c")
  sid = lax.axis_index("s")
  wid = sid * NC + cid
  r0 = sid * rows_per_tile

  pltpu.sync_copy(z_hbm.at[pl.ds(r0, rows_per_tile)],
                  acc_sh.at[pl.ds(r0, rows_per_tile)])
  if compute_deg:
    pltpu.sync_copy(z1_hbm.at[0], deg_v)
  plsc.subcore_barrier()

  ones16 = jnp.full((LANES,), 1.0, jnp.float32)
  nsub = (C + LANES - 1) // LANES

  def deg_count(j):
    if compute_deg:
      for k in range(nsub):
        lo = k * LANES
        if lo + LANES <= C:
          idx = dst_sv[j, pl.ds(lo, LANES)]
          plsc.addupdate_scatter(deg_v, [idx], ones16)
        else:
          idx = dst_sv[j, pl.ds(C - LANES, LANES)]
          mask = jax.lax.iota(jnp.int32, LANES) >= (lo - (C - LANES))
          plsc.addupdate_scatter(deg_v, [idx], ones16, mask=mask)

  def wait_gather(b):
    pltpu.make_async_copy(g_hbm.at[src_sv.at[0]], rows[b], semg[b]).wait()

  def wait_scatter(b):
    pltpu.make_async_copy(rows[b], acc_sh.at[dst_sv.at[0]], sems[b]).wait()

  def group(gi, _):
    pltpu.sync_copy(ei_hbm.at[0, wid, pl.ds(gi * G, G)], src_sv)
    pltpu.sync_copy(ei_hbm.at[1, wid, pl.ds(gi * G, G)], dst_sv)
    pltpu.async_copy(g_hbm.at[src_sv.at[0]], rows[0], semg[0])
    pltpu.async_copy(g_hbm.at[src_sv.at[1]], rows[1], semg[1])

    def pair(i, _):
      j0 = 2 * i
      wait_gather(0)
      pltpu.async_copy(rows[0], acc_sh.at[dst_sv.at[j0]], sems[0], add=True)
      wait_gather(1)
      pltpu.async_copy(rows[1], acc_sh.at[dst_sv.at[j0 + 1]], sems[1],
                       add=True)
      deg_count(j0)
      deg_count(j0 + 1)
      wait_scatter(0)
      pltpu.async_copy(g_hbm.at[src_sv.at[j0 + 2]], rows[0], semg[0])
      wait_scatter(1)
      pltpu.async_copy(g_hbm.at[src_sv.at[j0 + 3]], rows[1], semg[1])
      return 0
    lax.fori_loop(0, G // 2 - 1, pair, 0)
    wait_gather(0)
    pltpu.sync_copy(rows[0], acc_sh.at[dst_sv.at[G - 2]], add=True)
    wait_gather(1)
    pltpu.sync_copy(rows[1], acc_sh.at[dst_sv.at[G - 1]], add=True)
    deg_count(G - 2)
    deg_count(G - 1)
    return 0
  lax.fori_loop(0, NCH // G, group, 0)

  plsc.subcore_barrier()
  pltpu.sync_copy(acc_sh.at[pl.ds(r0, rows_per_tile)],
                  acc_out.at[pl.ds(cid * NP + r0, rows_per_tile)])
  if compute_deg:
    pltpu.sync_copy(deg_v, deg_out.at[wid])


def _sc_edge_pass(g, ei4, zerosD, zeros1, compute_deg):
  NP, D = g.shape
  NCH = ei4.shape[2]
  rows_per_tile = NP // NS
  mesh = plsc.VectorSubcoreMesh(core_axis_name="c", subcore_axis_name="s",
                                num_cores=NC, num_subcores=NS)
  out_type = [jax.ShapeDtypeStruct((NC * NP, D), jnp.float32)]
  scratch = [
      pltpu.VMEM((G, C), jnp.int32),
      pltpu.VMEM((G, C), jnp.int32),
      pltpu.VMEM((C, D), jnp.float32),
      pltpu.VMEM((C, D), jnp.float32),
      pltpu.VMEM_SHARED((NP, D), jnp.float32),
  ]
  if compute_deg:
    out_type.append(jax.ShapeDtypeStruct((NW, NP), jnp.float32))
    scratch.append(pltpu.VMEM((NP,), jnp.float32))
  scratch.extend([pltpu.SemaphoreType.DMA] * 4)
  body = functools.partial(_sc_edge_body, NP, D, NCH, rows_per_tile,
                           compute_deg)
  res = pl.kernel(
      body, out_type=tuple(out_type), mesh=mesh, scratch_types=scratch,
      compiler_params=pltpu.CompilerParams(needs_layout_passes=False),
  )(g, ei4, zerosD, zeros1)
  return res if compute_deg else (res[0], None)


def kernel(x, node_type, edge_index, f_W, f_b, W, b):
  N, D = x.shape
  L = W.shape[0]
  E = edge_index.shape[1]
  NCH = E // (NW * C)
  assert NW * NCH * C == E and NCH % G == 0

  ei4 = edge_index.reshape(2, NW, NCH, C)

  NP = ((N + 8 * NS - 1) // (8 * NS)) * (8 * NS)
  zerosD = jnp.zeros((NP, D), jnp.float32)
  zeros1 = jnp.zeros((1, NP), jnp.float32)

  block = NP // NS
  g = _tc_pre(x, node_type, f_W, f_b, W[0], NP, block)

  dinv = None
  for l in range(L):
    acc, deg = _sc_edge_pass(g, ei4, zerosD, zeros1,
                             compute_deg=(l == 0))
    if l == 0:
      dinv = (1.0 / jnp.maximum(deg.sum(axis=0), 1.0)).reshape(NP, 1)
    W_next = W[l + 1] if l + 1 < L else None
    g = _tc_post(acc, dinv, b[l], W_next, block)
  return g[:N]


# speedup vs baseline: 1.2091x; 1.2091x over previous
import functools

import jax
import jax.numpy as jnp
from jax import lax
from jax.experimental import pallas as pl
from jax.experimental.pallas import tpu as pltpu
from jax.experimental.pallas import tpu_sc as plsc

NC = 2
NS = 16
NW = NC * NS
LANES = 16
C = 125
G = 16


def _tc_pre_body(T, x_ref, nt_ref, fW_ref, fb_ref, W0_ref, out_ref):
  xb = x_ref[...]
  nt = nt_ref[...]
  h = jnp.zeros_like(xb)
  for t in range(T):
    ht = jnp.dot(xb, fW_ref[t], preferred_element_type=jnp.float32)
    ht = ht + fb_ref[t]
    h = jnp.where(nt == t, ht, h)
  out_ref[...] = jnp.dot(h, W0_ref[...], preferred_element_type=jnp.float32)


def _tc_pre(x, node_type, f_W, f_b, W0, NP, block):
  N, D = x.shape
  T = f_W.shape[0]
  grid = (NP // block,)
  return pl.pallas_call(
      functools.partial(_tc_pre_body, T),
      grid=grid,
      in_specs=[
          pl.BlockSpec((block, D), lambda i: (i, 0)),
          pl.BlockSpec((block, 1), lambda i: (i, 0)),
          pl.BlockSpec((T, D, D), lambda i: (0, 0, 0)),
          pl.BlockSpec((T, 1, D), lambda i: (0, 0, 0)),
          pl.BlockSpec((D, D), lambda i: (0, 0)),
      ],
      out_specs=pl.BlockSpec((block, D), lambda i: (i, 0)),
      out_shape=jax.ShapeDtypeStruct((NP, D), jnp.float32),
  )(x, node_type.reshape(N, 1), f_W, f_b.reshape(T, 1, D), W0)


def _tc_post_body(has_W, a0_ref, a1_ref, dinv_ref, b_ref, W_ref, out_ref):
  agg = a0_ref[...] + a1_ref[...]
  h = jnp.maximum(agg * dinv_ref[...] + b_ref[...], 0.0)
  if has_W:
    h = jnp.dot(h, W_ref[...], preferred_element_type=jnp.float32)
  out_ref[...] = h


def _tc_post(acc, dinv, b, W_next, block, out_rows=None):
  NP2, D = acc.shape
  NP = NP2 // 2
  grid = (NP // block,)
  nblk = NP // block
  if out_rows is None:
    out_rows = NP
  has_W = W_next is not None
  if W_next is None:
    W_next = jnp.zeros((8, 128), jnp.float32)
  WD = W_next.shape[0]
  return pl.pallas_call(
      functools.partial(_tc_post_body, has_W),
      grid=grid,
      in_specs=[
          pl.BlockSpec((block, D), lambda i: (i, 0)),
          pl.BlockSpec((block, D), lambda i: (i + nblk, 0)),
          pl.BlockSpec((block, 1), lambda i: (i, 0)),
          pl.BlockSpec((1, D), lambda i: (0, 0)),
          pl.BlockSpec((WD, D), lambda i: (0, 0)),
      ],
      out_specs=pl.BlockSpec((block, D), lambda i: (i, 0)),
      out_shape=jax.ShapeDtypeStruct((out_rows, D), jnp.float32),
  )(acc, acc, dinv, b.reshape(1, D), W_next)


def _sc_edge_body(NP, D, NCH, rows_per_tile, compute_deg,
                  g_hbm, ei_hbm, z_hbm, z1_hbm,
                  *refs):
  if compute_deg:
    (acc_out, deg_out, src_sv, dst_sv, rows_v0, rows_v1, acc_sh, deg_v,
     semg0, semg1, sems0, sems1) = refs
  else:
    (acc_out, src_sv, dst_sv, rows_v0, rows_v1, acc_sh,
     semg0, semg1, sems0, sems1) = refs
    deg_out = deg_v = None
  rows = (rows_v0, rows_v1)
  semg = (semg0, semg1)
  sems = (sems0, sems1)
  cid = lax.axis_index("c")
  sid = lax.axis_index("s")
  wid = sid * NC + cid
  r0 = sid * rows_per_tile

  pltpu.sync_copy(z_hbm.at[pl.ds(r0, rows_per_tile)],
                  acc_sh.at[pl.ds(r0, rows_per_tile)])
  if compute_deg:
    pltpu.sync_copy(z1_hbm.at[0], deg_v)
  plsc.subcore_barrier()

  ones16 = jnp.full((LANES,), 1.0, jnp.float32)
  nsub = (C + LANES - 1) // LANES

  def deg_count(j):
    if compute_deg:
      for k in range(nsub):
        lo = k * LANES
        if lo + LANES <= C:
          idx = dst_sv[j, pl.ds(lo, LANES)]
          plsc.addupdate_scatter(deg_v, [idx], ones16)
        else:
          idx = dst_sv[j, pl.ds(C - LANES, LANES)]
          mask = jax.lax.iota(jnp.int32, LANES) >= (lo - (C - LANES))
          plsc.addupdate_scatter(deg_v, [idx], ones16, mask=mask)

  def group(gi, _):
    pltpu.sync_copy(ei_hbm.at[0, wid, pl.ds(gi * G, G)], src_sv)
    pltpu.sync_copy(ei_hbm.at[1, wid, pl.ds(gi * G, G)], dst_sv)
    pltpu.async_copy(g_hbm.at[src_sv.at[0]], rows[0], semg[0])

    def pair(i, _):
      j0 = 2 * i
      pltpu.async_copy(g_hbm.at[src_sv.at[j0 + 1]], rows[1], semg[1])
      pltpu.make_async_copy(g_hbm.at[src_sv.at[j0]], rows[0], semg[0]).wait()
      pltpu.sync_copy(rows[0], acc_sh.at[dst_sv.at[j0]], add=True)
      deg_count(j0)

      @pl.when(j0 + 2 < G)
      def _():
        pltpu.async_copy(g_hbm.at[src_sv.at[j0 + 2]], rows[0], semg[0])
      pltpu.make_async_copy(
          g_hbm.at[src_sv.at[j0 + 1]], rows[1], semg[1]).wait()
      pltpu.sync_copy(rows[1], acc_sh.at[dst_sv.at[j0 + 1]], add=True)
      deg_count(j0 + 1)
      return 0
    lax.fori_loop(0, G // 2, pair, 0)
    return 0
  lax.fori_loop(0, NCH // G, group, 0)

  plsc.subcore_barrier()
  pltpu.sync_copy(acc_sh.at[pl.ds(r0, rows_per_tile)],
                  acc_out.at[pl.ds(cid * NP + r0, rows_per_tile)])
  if compute_deg:
    pltpu.sync_copy(deg_v, deg_out.at[wid])


def _sc_edge_pass(g, ei4, zerosD, zeros1, compute_deg):
  NP, D = g.shape
  NCH = ei4.shape[2]
  rows_per_tile = NP // NS
  mesh = plsc.VectorSubcoreMesh(core_axis_name="c", subcore_axis_name="s",
                                num_cores=NC, num_subcores=NS)
  out_type = [jax.ShapeDtypeStruct((NC * NP, D), jnp.float32)]
  scratch = [
      pltpu.VMEM((G, C), jnp.int32),
      pltpu.VMEM((G, C), jnp.int32),
      pltpu.VMEM((C, D), jnp.float32),
      pltpu.VMEM((C, D), jnp.float32),
      pltpu.VMEM_SHARED((NP, D), jnp.float32),
  ]
  if compute_deg:
    out_type.append(jax.ShapeDtypeStruct((NW, NP), jnp.float32))
    scratch.append(pltpu.VMEM((NP,), jnp.float32))
  scratch.extend([pltpu.SemaphoreType.DMA] * 4)
  body = functools.partial(_sc_edge_body, NP, D, NCH, rows_per_tile,
                           compute_deg)
  res = pl.kernel(
      body, out_type=tuple(out_type), mesh=mesh, scratch_types=scratch,
      compiler_params=pltpu.CompilerParams(needs_layout_passes=False),
  )(g, ei4, zerosD, zeros1)
  return res if compute_deg else (res[0], None)


def kernel(x, node_type, edge_index, f_W, f_b, W, b):
  N, D = x.shape
  L = W.shape[0]
  E = edge_index.shape[1]
  NCH = E // (NW * C)
  assert NW * NCH * C == E and NCH % G == 0

  ei4 = edge_index.reshape(2, NW, NCH, C)

  NP = ((N + 8 * NS - 1) // (8 * NS)) * (8 * NS)
  zerosD = jnp.zeros((NP, D), jnp.float32)
  zeros1 = jnp.zeros((1, NP), jnp.float32)

  block = NP // NS
  g = _tc_pre(x, node_type, f_W, f_b, W[0], NP, block)

  dinv = None
  for l in range(L):
    acc, deg = _sc_edge_pass(g, ei4, zerosD, zeros1,
                             compute_deg=(l == 0))
    if l == 0:
      dinv = (1.0 / jnp.maximum(deg.sum(axis=0), 1.0)).reshape(NP, 1)
    W_next = W[l + 1] if l + 1 < L else None
    out_rows = NP if l + 1 < L else N
    g = _tc_post(acc, dinv, b[l], W_next, block, out_rows)
  return g
